# baseline (speedup 1.0000x reference)
import jax
import jax.numpy as jnp
from jax import lax
from jax.experimental import pallas as pl
from jax.experimental.pallas import tpu as pltpu
from jax.experimental.pallas import tpu_sc as plsc

NV = 10000
NRINGS = 8
NDIRS = 8
NCH = 16
NFILTERS = 16

VCHUNK = 128
NFULL = NV // VCHUNK
VTAIL = NV - NFULL * VCHUNK
ZCOLS = 260
NVPAD = 10240


def _sc_gather_body(yq_hbm, ctr_hbm, w_hbm, z_hbm, y_v, ctr_v, w_v, out_v,
                    sc0, sc1, sw0, sw1, so):
    h = lax.axis_index("c")
    s = lax.axis_index("s")
    q = 2 * h + lax.rem(s, 2)
    g = lax.div(s, 2)

    pltpu.sync_copy(yq_hbm.at[q], y_v)

    rowq = [jnp.broadcast_to(jnp.int32(ct), (16,)) for ct in range(4)]
    semc = [sc0, sc1]
    semw = [sw0, sw1]

    def start(v0, r, b, L=VCHUNK):
        if L == VCHUNK:
            cdst, wdst = ctr_v.at[b], w_v.at[b]
        else:
            cdst = ctr_v.at[b, :, :, pl.ds(0, L)]
            wdst = w_v.at[b, :, :, pl.ds(0, L)]
        hc = pltpu.async_copy(
            ctr_hbm.at[pl.ds(3 * r, 3), :, pl.ds(v0, L)], cdst, semc[b])
        hw = pltpu.async_copy(
            w_hbm.at[pl.ds(3 * r, 3), :, pl.ds(v0, L)], wdst, semw[b])
        return hc, hw

    def compute_r(r, b, ob, ngroups):
        @plsc.parallel_loop(0, 8 * ngroups, unroll=4)
        def _(dg):
            d = lax.div(dg, ngroups)
            gg = lax.rem(dg, ngroups)
            base = gg * 16
            c0 = ctr_v[b, 0, d, pl.ds(base, 16)]
            c1 = ctr_v[b, 1, d, pl.ds(base, 16)]
            c2 = ctr_v[b, 2, d, pl.ds(base, 16)]
            w0 = w_v[b, 0, d, pl.ds(base, 16)]
            w1 = w_v[b, 1, d, pl.ds(base, 16)]
            w2 = w_v[b, 2, d, pl.ds(base, 16)]
            colbase = r * 32 + d * 4
            for ct in range(4):
                g0 = plsc.load_gather(y_v, [rowq[ct], c0])
                g1 = plsc.load_gather(y_v, [rowq[ct], c1])
                g2 = plsc.load_gather(y_v, [rowq[ct], c2])
                acc = w0 * g0 + w1 * g1 + w2 * g2
                ob[colbase + ct, pl.ds(base, 16)] = acc

    def center_cols(v0, ob, ngroups):
        @plsc.parallel_loop(0, ngroups)
        def _(gg):
            for ct in range(4):
                ob[256 + ct, pl.ds(gg * 16, 16)] = (
                    y_v[ct, pl.ds(v0 + gg * 16, 16)])

    def chunk_body(t, carry):
        ci = g + 8 * t

        @pl.when(ci < NFULL)
        def _():
            v0 = pl.multiple_of(ci * VCHUNK, VCHUNK)

            @pl.when(t == 0)
            def _():
                hc, hw = start(v0, 0, 0)
                hc.wait()
                hw.wait()

            @pl.when(t > 0)
            def _():
                pltpu.make_async_copy(
                    ctr_hbm.at[pl.ds(0, 3), :, pl.ds(0, VCHUNK)],
                    ctr_v.at[0], semc[0]).wait()
                pltpu.make_async_copy(
                    w_hbm.at[pl.ds(0, 3), :, pl.ds(0, VCHUNK)],
                    w_v.at[0], semw[0]).wait()
                pltpu.make_async_copy(
                    out_v, z_hbm.at[q, :, pl.ds(0, VCHUNK)], so).wait()

            handles = [None, None]
            for r in range(NRINGS):
                b = r & 1
                if r + 1 < NRINGS:
                    handles[1 - b] = start(v0, r + 1, 1 - b)
                if r > 0:
                    handles[b][0].wait()
                    handles[b][1].wait()
                if r == NRINGS - 1:
                    @pl.when(ci + 8 < NFULL)
                    def _():
                        start(pl.multiple_of((ci + 8) * VCHUNK, VCHUNK), 0, 0)
                compute_r(r, b, out_v, 8)

            center_cols(v0, out_v, 8)

            pltpu.async_copy(out_v, z_hbm.at[q, :, pl.ds(v0, VCHUNK)], so)
        return carry

    lax.fori_loop(0, -(-NFULL // 8), chunk_body, 0)

    pltpu.make_async_copy(out_v, z_hbm.at[q, :, pl.ds(0, VCHUNK)],
                          so).wait()

    @pl.when(g == 7)
    def _():
        v0 = pl.multiple_of(NFULL * VCHUNK, VCHUNK)
        ob = out_v
        handles = [None, None]
        handles[0] = start(v0, 0, 0, VTAIL)
        for r in range(NRINGS):
            b = r & 1
            if r + 1 < NRINGS:
                handles[1 - b] = start(v0, r + 1, 1 - b, VTAIL)
            handles[b][0].wait()
            handles[b][1].wait()
            compute_r(r, b, ob, 1)
        center_cols(v0, ob, 1)
        pltpu.sync_copy(out_v, z_hbm.at[q, :, pl.ds(v0, VCHUNK)])


@jax.jit
def _sc_gather(y_q, ctr_t, w_t):
    mesh = plsc.VectorSubcoreMesh(core_axis_name="c", subcore_axis_name="s")
    f = pl.kernel(
        _sc_gather_body,
        out_type=jax.ShapeDtypeStruct((4, ZCOLS, NVPAD), jnp.float32),
        mesh=mesh,
        scratch_types=[
            pltpu.VMEM((4, NV), jnp.float32),
            pltpu.VMEM((2, 3, 8, VCHUNK), jnp.int32),
            pltpu.VMEM((2, 3, 8, VCHUNK), jnp.float32),
            pltpu.VMEM((ZCOLS, VCHUNK), jnp.float32),
            pltpu.SemaphoreType.DMA,
            pltpu.SemaphoreType.DMA,
            pltpu.SemaphoreType.DMA,
            pltpu.SemaphoreType.DMA,
            pltpu.SemaphoreType.DMA,
        ],
        compiler_params=pltpu.CompilerParams(needs_layout_passes=False),
    )
    return f(y_q, ctr_t, w_t)


V_BLK = 1280


def _tc_body(z_ref, kb_ref, bb_ref, ob_ref):
    zb = z_ref[...]
    kb = kb_ref[...]
    acc = jnp.dot(kb[0], zb[0], preferred_element_type=jnp.float32)
    for qq in range(1, 4):
        acc = acc + jnp.dot(kb[qq], zb[qq],
                            preferred_element_type=jnp.float32)
    m = acc[0:16]
    for dd in range(1, 8):
        m = jnp.maximum(m, acc[dd * 16:(dd + 1) * 16])
    ob_ref[...] = jnp.maximum(m + bb_ref[...], 0.0)


@jax.jit
def _tc_conv(z, KbigT, bias2):
    grid = (NVPAD // V_BLK,)
    return pl.pallas_call(
        _tc_body,
        grid=grid,
        in_specs=[
            pl.BlockSpec((4, ZCOLS, V_BLK), lambda i: (0, 0, i)),
            pl.BlockSpec((4, 128, ZCOLS), lambda i: (0, 0, 0)),
            pl.BlockSpec((NFILTERS, 1), lambda i: (0, 0)),
        ],
        out_specs=pl.BlockSpec((NFILTERS, V_BLK), lambda i: (0, i)),
        out_shape=jax.ShapeDtypeStruct((NFILTERS, NVPAD), jnp.float32),
    )(z, KbigT, bias2)


def kernel(y, contributors, weights, angles, kernel, center_kernel, bias):
    del angles

    y_q = jnp.transpose(y, (0, 2, 1)).reshape(4, 4, NV)
    ctr_t = jnp.transpose(contributors, (0, 2, 4, 3, 1)).reshape(24, 8, NV)
    w_t = jnp.transpose(weights, (0, 2, 4, 3, 1)).reshape(24, 8, NV)

    Kb = jnp.stack([jnp.roll(kernel, dd, axis=1) for dd in range(NDIRS)],
                   axis=-2)
    Kb = Kb.reshape(NRINGS, NDIRS, 4, 4, NDIRS * NFILTERS)
    Kb = jnp.transpose(Kb, (2, 0, 1, 3, 4)).reshape(4, 256, 128)
    ckt = jnp.tile(center_kernel.reshape(4, 4, 1, NFILTERS),
                   (1, 1, NDIRS, 1)).reshape(4, 4, 128)
    Kbig = jnp.concatenate([Kb, ckt], axis=1)
    KbigT = jnp.transpose(Kbig, (0, 2, 1))

    z = _sc_gather(y_q, ctr_t, w_t)
    outT = _tc_conv(z, KbigT, bias.reshape(NFILTERS, 1))
    return jnp.transpose(outT[:, :NV])[None]


# speedup vs baseline: 1.0181x; 1.0181x over previous
import jax
import jax.numpy as jnp
from jax import lax
from jax.experimental import pallas as pl
from jax.experimental.pallas import tpu as pltpu
from jax.experimental.pallas import tpu_sc as plsc

NV = 10000
NRINGS = 8
NDIRS = 8
NCH = 16
NFILTERS = 16

VCHUNK = 128
NFULL = NV // VCHUNK
VTAIL = NV - NFULL * VCHUNK
ZCOLS = 260
NVPAD = 10240


def _sc_gather_body(yq_hbm, ctr_hbm, w_hbm, z_hbm, y_v, ctr_v, w_v, out_v,
                    sc0, sc1, sw0, sw1, so):
    h = lax.axis_index("c")
    s = lax.axis_index("s")
    q = 2 * h + lax.rem(s, 2)
    g = lax.div(s, 2)

    pltpu.sync_copy(yq_hbm.at[q], y_v)

    rowq = [jnp.broadcast_to(jnp.int32(ct), (16,)) for ct in range(4)]
    semc = [sc0, sc1]
    semw = [sw0, sw1]

    def start(v0, r, b, L=VCHUNK):
        if L == VCHUNK:
            cdst, wdst = ctr_v.at[b], w_v.at[b]
        else:
            cdst = ctr_v.at[b, :, :, pl.ds(0, L)]
            wdst = w_v.at[b, :, :, pl.ds(0, L)]
        hc = pltpu.async_copy(
            ctr_hbm.at[pl.ds(3 * r, 3), :, pl.ds(v0, L)], cdst, semc[b])
        hw = pltpu.async_copy(
            w_hbm.at[pl.ds(3 * r, 3), :, pl.ds(v0, L)], wdst, semw[b])
        return hc, hw

    def compute_r(r, b, ob, ngroups):
        @plsc.parallel_loop(0, 8 * ngroups, unroll=2)
        def _(dg):
            d = lax.div(dg, ngroups)
            gg = lax.rem(dg, ngroups)
            base = gg * 16
            c0 = ctr_v[b, 0, d, pl.ds(base, 16)]
            c1 = ctr_v[b, 1, d, pl.ds(base, 16)]
            c2 = ctr_v[b, 2, d, pl.ds(base, 16)]
            w0 = w_v[b, 0, d, pl.ds(base, 16)]
            w1 = w_v[b, 1, d, pl.ds(base, 16)]
            w2 = w_v[b, 2, d, pl.ds(base, 16)]
            colbase = r * 32 + d * 4
            for ct in range(4):
                g0 = plsc.load_gather(y_v, [rowq[ct], c0])
                g1 = plsc.load_gather(y_v, [rowq[ct], c1])
                g2 = plsc.load_gather(y_v, [rowq[ct], c2])
                acc = w0 * g0 + w1 * g1 + w2 * g2
                ob[colbase + ct, pl.ds(base, 16)] = acc

    def center_cols(v0, ob, ngroups):
        @plsc.parallel_loop(0, ngroups)
        def _(gg):
            for ct in range(4):
                ob[256 + ct, pl.ds(gg * 16, 16)] = (
                    y_v[ct, pl.ds(v0 + gg * 16, 16)])

    def chunk_body(t, carry):
        ci = g + 8 * t

        @pl.when(ci < NFULL)
        def _():
            v0 = pl.multiple_of(ci * VCHUNK, VCHUNK)

            @pl.when(t == 0)
            def _():
                hc, hw = start(v0, 0, 0)
                hc.wait()
                hw.wait()

            @pl.when(t > 0)
            def _():
                pltpu.make_async_copy(
                    ctr_hbm.at[pl.ds(0, 3), :, pl.ds(0, VCHUNK)],
                    ctr_v.at[0], semc[0]).wait()
                pltpu.make_async_copy(
                    w_hbm.at[pl.ds(0, 3), :, pl.ds(0, VCHUNK)],
                    w_v.at[0], semw[0]).wait()
                pltpu.make_async_copy(
                    out_v, z_hbm.at[q, :, pl.ds(0, VCHUNK)], so).wait()

            handles = [None, None]
            for r in range(NRINGS):
                b = r & 1
                if r + 1 < NRINGS:
                    handles[1 - b] = start(v0, r + 1, 1 - b)
                if r > 0:
                    handles[b][0].wait()
                    handles[b][1].wait()
                if r == NRINGS - 1:
                    @pl.when(ci + 8 < NFULL)
                    def _():
                        start(pl.multiple_of((ci + 8) * VCHUNK, VCHUNK), 0, 0)
                compute_r(r, b, out_v, 8)

            center_cols(v0, out_v, 8)

            pltpu.async_copy(out_v, z_hbm.at[q, :, pl.ds(v0, VCHUNK)], so)
        return carry

    lax.fori_loop(0, -(-NFULL // 8), chunk_body, 0)

    pltpu.make_async_copy(out_v, z_hbm.at[q, :, pl.ds(0, VCHUNK)],
                          so).wait()

    @pl.when(g == 7)
    def _():
        v0 = pl.multiple_of(NFULL * VCHUNK, VCHUNK)
        ob = out_v
        handles = [None, None]
        handles[0] = start(v0, 0, 0, VTAIL)
        for r in range(NRINGS):
            b = r & 1
            if r + 1 < NRINGS:
                handles[1 - b] = start(v0, r + 1, 1 - b, VTAIL)
            handles[b][0].wait()
            handles[b][1].wait()
            compute_r(r, b, ob, 1)
        center_cols(v0, ob, 1)
        pltpu.sync_copy(out_v, z_hbm.at[q, :, pl.ds(v0, VCHUNK)])


@jax.jit
def _sc_gather(y_q, ctr_t, w_t):
    mesh = plsc.VectorSubcoreMesh(core_axis_name="c", subcore_axis_name="s")
    f = pl.kernel(
        _sc_gather_body,
        out_type=jax.ShapeDtypeStruct((4, ZCOLS, NVPAD), jnp.float32),
        mesh=mesh,
        scratch_types=[
            pltpu.VMEM((4, NV), jnp.float32),
            pltpu.VMEM((2, 3, 8, VCHUNK), jnp.int32),
            pltpu.VMEM((2, 3, 8, VCHUNK), jnp.float32),
            pltpu.VMEM((ZCOLS, VCHUNK), jnp.float32),
            pltpu.SemaphoreType.DMA,
            pltpu.SemaphoreType.DMA,
            pltpu.SemaphoreType.DMA,
            pltpu.SemaphoreType.DMA,
            pltpu.SemaphoreType.DMA,
        ],
        compiler_params=pltpu.CompilerParams(needs_layout_passes=False),
    )
    return f(y_q, ctr_t, w_t)


V_BLK = 1280


def _tc_body(z_ref, kb_ref, bb_ref, ob_ref):
    zb = z_ref[...]
    kb = kb_ref[...]
    acc = jnp.dot(kb[0], zb[0], preferred_element_type=jnp.float32)
    for qq in range(1, 4):
        acc = acc + jnp.dot(kb[qq], zb[qq],
                            preferred_element_type=jnp.float32)
    m = acc[0:16]
    for dd in range(1, 8):
        m = jnp.maximum(m, acc[dd * 16:(dd + 1) * 16])
    ob_ref[...] = jnp.transpose(jnp.maximum(m + bb_ref[...], 0.0))


@jax.jit
def _tc_conv(z, KbigT, bias2):
    grid = (NVPAD // V_BLK,)
    return pl.pallas_call(
        _tc_body,
        grid=grid,
        in_specs=[
            pl.BlockSpec((4, ZCOLS, V_BLK), lambda i: (0, 0, i)),
            pl.BlockSpec((4, 128, ZCOLS), lambda i: (0, 0, 0)),
            pl.BlockSpec((NFILTERS, 1), lambda i: (0, 0)),
        ],
        out_specs=pl.BlockSpec((V_BLK, NFILTERS), lambda i: (i, 0)),
        out_shape=jax.ShapeDtypeStruct((NVPAD, NFILTERS), jnp.float32),
    )(z, KbigT, bias2)


def kernel(y, contributors, weights, angles, kernel, center_kernel, bias):
    del angles

    y_q = jnp.transpose(y, (0, 2, 1)).reshape(4, 4, NV)
    ctr_t = jnp.transpose(contributors, (0, 2, 4, 3, 1)).reshape(24, 8, NV)
    w_t = jnp.transpose(weights, (0, 2, 4, 3, 1)).reshape(24, 8, NV)

    Kb = jnp.stack([jnp.roll(kernel, dd, axis=1) for dd in range(NDIRS)],
                   axis=-2)
    Kb = Kb.reshape(NRINGS, NDIRS, 4, 4, NDIRS * NFILTERS)
    Kb = jnp.transpose(Kb, (2, 0, 1, 3, 4)).reshape(4, 256, 128)
    ckt = jnp.tile(center_kernel.reshape(4, 4, 1, NFILTERS),
                   (1, 1, NDIRS, 1)).reshape(4, 4, 128)
    Kbig = jnp.concatenate([Kb, ckt], axis=1)
    KbigT = jnp.transpose(Kbig, (0, 2, 1))

    z = _sc_gather(y_q, ctr_t, w_t)
    out = _tc_conv(z, KbigT, bias.reshape(NFILTERS, 1))
    return out[:NV][None]


# speedup vs baseline: 1.0533x; 1.0345x over previous
import jax
import jax.numpy as jnp
from jax import lax
from jax.experimental import pallas as pl
from jax.experimental.pallas import tpu as pltpu
from jax.experimental.pallas import tpu_sc as plsc

NV = 10000
NRINGS = 8
NDIRS = 8
NCH = 16
NFILTERS = 16

VCHUNK = 128
NFULL = NV // VCHUNK
VTAIL = NV - NFULL * VCHUNK
ZCOLS = 260
NVPAD = 10240


def _sc_gather_body(yq_hbm, ctr_hbm, w_hbm, z_hbm, y_v, ctr_v, w_v, out_v,
                    sc0, sc1, sw0, sw1, so):
    h = lax.axis_index("c")
    s = lax.axis_index("s")
    q = 2 * h + lax.rem(s, 2)
    g = lax.div(s, 2)

    pltpu.sync_copy(yq_hbm.at[q], y_v)

    rowq = [jnp.broadcast_to(jnp.int32(ct), (16,)) for ct in range(4)]
    semc = [sc0, sc1]
    semw = [sw0, sw1]

    def start(v0, r, b, L=VCHUNK):
        if L == VCHUNK:
            cdst, wdst = ctr_v.at[b], w_v.at[b]
        else:
            cdst = ctr_v.at[b, :, :, pl.ds(0, L)]
            wdst = w_v.at[b, :, :, pl.ds(0, L)]
        hc = pltpu.async_copy(
            ctr_hbm.at[pl.ds(3 * r, 3), :, pl.ds(v0, L)], cdst, semc[b])
        hw = pltpu.async_copy(
            w_hbm.at[pl.ds(3 * r, 3), :, pl.ds(v0, L)], wdst, semw[b])
        return hc, hw

    def compute_r(r, b, ob, ngroups):
        @plsc.parallel_loop(0, 8 * ngroups, unroll=2)
        def _(dg):
            d = lax.div(dg, ngroups)
            gg = lax.rem(dg, ngroups)
            base = gg * 16
            c0 = ctr_v[b, 0, d, pl.ds(base, 16)]
            c1 = ctr_v[b, 1, d, pl.ds(base, 16)]
            c2 = ctr_v[b, 2, d, pl.ds(base, 16)]
            w0 = w_v[b, 0, d, pl.ds(base, 16)]
            w1 = w_v[b, 1, d, pl.ds(base, 16)]
            w2 = w_v[b, 2, d, pl.ds(base, 16)]
            colbase = r * 32 + d * 4
            for ct in range(4):
                g0 = plsc.load_gather(y_v, [rowq[ct], c0])
                g1 = plsc.load_gather(y_v, [rowq[ct], c1])
                g2 = plsc.load_gather(y_v, [rowq[ct], c2])
                acc = w0 * g0 + w1 * g1 + w2 * g2
                ob[colbase + ct, pl.ds(base, 16)] = acc

    def center_cols(v0, ob, ngroups):
        @plsc.parallel_loop(0, ngroups)
        def _(gg):
            for ct in range(4):
                ob[256 + ct, pl.ds(gg * 16, 16)] = (
                    y_v[ct, pl.ds(v0 + gg * 16, 16)])

    def chunk_body(t, carry):
        ci = g + 8 * t

        @pl.when(ci < NFULL)
        def _():
            v0 = pl.multiple_of(ci * VCHUNK, VCHUNK)

            @pl.when(t == 0)
            def _():
                hc, hw = start(v0, 0, 0)
                hc.wait()
                hw.wait()

            @pl.when(t > 0)
            def _():
                pltpu.make_async_copy(
                    ctr_hbm.at[pl.ds(0, 3), :, pl.ds(0, VCHUNK)],
                    ctr_v.at[0], semc[0]).wait()
                pltpu.make_async_copy(
                    w_hbm.at[pl.ds(0, 3), :, pl.ds(0, VCHUNK)],
                    w_v.at[0], semw[0]).wait()
                pltpu.make_async_copy(
                    out_v, z_hbm.at[q, :, pl.ds(0, VCHUNK)], so).wait()

            handles = [None, None]
            for r in range(NRINGS):
                b = r & 1
                if r + 1 < NRINGS:
                    handles[1 - b] = start(v0, r + 1, 1 - b)
                if r > 0:
                    handles[b][0].wait()
                    handles[b][1].wait()
                if r == NRINGS - 1:
                    @pl.when(ci + 8 < NFULL)
                    def _():
                        start(pl.multiple_of((ci + 8) * VCHUNK, VCHUNK), 0, 0)
                compute_r(r, b, out_v, 8)

            center_cols(v0, out_v, 8)

            pltpu.async_copy(out_v, z_hbm.at[q, :, pl.ds(v0, VCHUNK)], so)
        return carry

    lax.fori_loop(0, -(-NFULL // 8), chunk_body, 0)

    pltpu.make_async_copy(out_v, z_hbm.at[q, :, pl.ds(0, VCHUNK)],
                          so).wait()

    @pl.when(g == 7)
    def _():
        v0 = pl.multiple_of(NFULL * VCHUNK, VCHUNK)
        ob = out_v
        handles = [None, None]
        handles[0] = start(v0, 0, 0, VTAIL)
        for r in range(NRINGS):
            b = r & 1
            if r + 1 < NRINGS:
                handles[1 - b] = start(v0, r + 1, 1 - b, VTAIL)
            handles[b][0].wait()
            handles[b][1].wait()
            compute_r(r, b, ob, 1)
        center_cols(v0, ob, 1)
        pltpu.sync_copy(out_v, z_hbm.at[q, :, pl.ds(v0, VCHUNK)])


@jax.jit
def _sc_gather(y_q, ctr_t, w_t):
    mesh = plsc.VectorSubcoreMesh(core_axis_name="c", subcore_axis_name="s")
    f = pl.kernel(
        _sc_gather_body,
        out_type=jax.ShapeDtypeStruct((4, ZCOLS, NVPAD), jnp.float32),
        mesh=mesh,
        scratch_types=[
            pltpu.VMEM((4, NV), jnp.float32),
            pltpu.VMEM((2, 3, 8, VCHUNK), jnp.int32),
            pltpu.VMEM((2, 3, 8, VCHUNK), jnp.float32),
            pltpu.VMEM((ZCOLS, VCHUNK), jnp.float32),
            pltpu.SemaphoreType.DMA,
            pltpu.SemaphoreType.DMA,
            pltpu.SemaphoreType.DMA,
            pltpu.SemaphoreType.DMA,
            pltpu.SemaphoreType.DMA,
        ],
        compiler_params=pltpu.CompilerParams(needs_layout_passes=False),
    )
    return f(y_q, ctr_t, w_t)


V_BLK = 1280


def _tc_body(z_ref, kb_ref, bb_ref, ob_ref):
    zb = z_ref[...].astype(jnp.bfloat16)
    kb = kb_ref[...]
    acc = jnp.dot(kb[0], zb[0], preferred_element_type=jnp.float32)
    for qq in range(1, 4):
        acc = acc + jnp.dot(kb[qq], zb[qq],
                            preferred_element_type=jnp.float32)
    m = acc[0:16]
    for dd in range(1, 8):
        m = jnp.maximum(m, acc[dd * 16:(dd + 1) * 16])
    ob_ref[...] = jnp.maximum(m + bb_ref[...], 0.0)


@jax.jit
def _tc_conv(z, KbigT, bias2):
    grid = (NVPAD // V_BLK,)
    return pl.pallas_call(
        _tc_body,
        grid=grid,
        in_specs=[
            pl.BlockSpec((4, ZCOLS, V_BLK), lambda i: (0, 0, i)),
            pl.BlockSpec((4, 128, ZCOLS), lambda i: (0, 0, 0)),
            pl.BlockSpec((NFILTERS, 1), lambda i: (0, 0)),
        ],
        out_specs=pl.BlockSpec((NFILTERS, V_BLK), lambda i: (0, i)),
        out_shape=jax.ShapeDtypeStruct((NFILTERS, NVPAD), jnp.float32),
    )(z, KbigT, bias2)


def kernel(y, contributors, weights, angles, kernel, center_kernel, bias):
    del angles

    y_q = jnp.transpose(y, (0, 2, 1)).reshape(4, 4, NV)
    ctr_t = jnp.transpose(contributors, (0, 2, 4, 3, 1)).reshape(24, 8, NV)
    w_t = jnp.transpose(weights, (0, 2, 4, 3, 1)).reshape(24, 8, NV)

    Kb = jnp.stack([jnp.roll(kernel, dd, axis=1) for dd in range(NDIRS)],
                   axis=-2)
    Kb = Kb.reshape(NRINGS, NDIRS, 4, 4, NDIRS * NFILTERS)
    Kb = jnp.transpose(Kb, (2, 0, 1, 3, 4)).reshape(4, 256, 128)
    ckt = jnp.tile(center_kernel.reshape(4, 4, 1, NFILTERS),
                   (1, 1, NDIRS, 1)).reshape(4, 4, 128)
    Kbig = jnp.concatenate([Kb, ckt], axis=1)
    KbigT = jnp.transpose(Kbig, (0, 2, 1)).astype(jnp.bfloat16)

    z = _sc_gather(y_q, ctr_t, w_t)
    outT = _tc_conv(z, KbigT, bias.reshape(NFILTERS, 1))
    return jnp.transpose(outT[:, :NV])[None]


# speedup vs baseline: 1.1115x; 1.0553x over previous
import jax
import jax.numpy as jnp
from jax import lax
from jax.experimental import pallas as pl
from jax.experimental.pallas import tpu as pltpu
from jax.experimental.pallas import tpu_sc as plsc

NV = 10000
NRINGS = 8
NDIRS = 8
NCH = 16
NFILTERS = 16

VCHUNK = 128
NFULL = NV // VCHUNK
VTAIL = NV - NFULL * VCHUNK
ZCOLS = 260
NVPAD = 10240


def _sc_gather_body(yq_hbm, ctr_hbm, w_hbm, z_hbm, y_v, ctr_v, w_v, out_v,
                    sc0, sc1, sw0, sw1, soA, soB):
    h = lax.axis_index("c")
    s = lax.axis_index("s")
    q = 2 * h + lax.rem(s, 2)
    g = lax.div(s, 2)

    pltpu.sync_copy(yq_hbm.at[q], y_v)

    rowq = [jnp.broadcast_to(jnp.int32(ct), (16,)) for ct in range(4)]
    semc = [sc0, sc1]
    semw = [sw0, sw1]

    def start(v0, r, b, L=VCHUNK):
        if L == VCHUNK:
            cdst, wdst = ctr_v.at[b], w_v.at[b]
        else:
            cdst = ctr_v.at[b, :, :, pl.ds(0, L)]
            wdst = w_v.at[b, :, :, pl.ds(0, L)]
        hc = pltpu.async_copy(
            ctr_hbm.at[pl.ds(3 * r, 3), :, pl.ds(v0, L)], cdst, semc[b])
        hw = pltpu.async_copy(
            w_hbm.at[pl.ds(3 * r, 3), :, pl.ds(v0, L)], wdst, semw[b])
        return hc, hw

    def compute_r(r, b, ob, ngroups):
        @plsc.parallel_loop(0, 8 * ngroups, unroll=2)
        def _(dg):
            d = lax.div(dg, ngroups)
            gg = lax.rem(dg, ngroups)
            base = gg * 16
            c0 = ctr_v[b, 0, d, pl.ds(base, 16)]
            c1 = ctr_v[b, 1, d, pl.ds(base, 16)]
            c2 = ctr_v[b, 2, d, pl.ds(base, 16)]
            w0 = w_v[b, 0, d, pl.ds(base, 16)]
            w1 = w_v[b, 1, d, pl.ds(base, 16)]
            w2 = w_v[b, 2, d, pl.ds(base, 16)]
            colbase = r * 32 + d * 4
            for ct in range(4):
                g0 = plsc.load_gather(y_v, [rowq[ct], c0])
                g1 = plsc.load_gather(y_v, [rowq[ct], c1])
                g2 = plsc.load_gather(y_v, [rowq[ct], c2])
                acc = w0 * g0 + w1 * g1 + w2 * g2
                ob[colbase + ct, pl.ds(base, 16)] = acc

    def center_cols(v0, ob, ngroups):
        @plsc.parallel_loop(0, ngroups)
        def _(gg):
            for ct in range(4):
                ob[256 + ct, pl.ds(gg * 16, 16)] = (
                    y_v[ct, pl.ds(v0 + gg * 16, 16)])

    def chunk_body(t, carry):
        ci = g + 8 * t

        @pl.when(ci < NFULL)
        def _():
            v0 = pl.multiple_of(ci * VCHUNK, VCHUNK)

            @pl.when(t == 0)
            def _():
                hc, hw = start(v0, 0, 0)
                hc.wait()
                hw.wait()

            @pl.when(t > 0)
            def _():
                pltpu.make_async_copy(
                    ctr_hbm.at[pl.ds(0, 3), :, pl.ds(0, VCHUNK)],
                    ctr_v.at[0], semc[0]).wait()
                pltpu.make_async_copy(
                    w_hbm.at[pl.ds(0, 3), :, pl.ds(0, VCHUNK)],
                    w_v.at[0], semw[0]).wait()
                pltpu.make_async_copy(
                    out_v.at[pl.ds(0, 128)],
                    z_hbm.at[q, pl.ds(0, 128), pl.ds(0, VCHUNK)],
                    soA).wait()

            handles = [None, None]
            for r in range(NRINGS):
                b = r & 1
                if r + 1 < NRINGS:
                    handles[1 - b] = start(v0, r + 1, 1 - b)
                if r > 0:
                    handles[b][0].wait()
                    handles[b][1].wait()
                if r == 4:
                    @pl.when(t > 0)
                    def _():
                        pltpu.make_async_copy(
                            out_v.at[pl.ds(128, ZCOLS - 128)],
                            z_hbm.at[q, pl.ds(128, ZCOLS - 128),
                                     pl.ds(0, VCHUNK)],
                            soB).wait()
                if r == NRINGS - 1:
                    @pl.when(ci + 8 < NFULL)
                    def _():
                        start(pl.multiple_of((ci + 8) * VCHUNK, VCHUNK), 0, 0)
                compute_r(r, b, out_v, 8)
                if r == 3:
                    pltpu.async_copy(
                        out_v.at[pl.ds(0, 128)],
                        z_hbm.at[q, pl.ds(0, 128), pl.ds(v0, VCHUNK)], soA)

            center_cols(v0, out_v, 8)

            pltpu.async_copy(
                out_v.at[pl.ds(128, ZCOLS - 128)],
                z_hbm.at[q, pl.ds(128, ZCOLS - 128), pl.ds(v0, VCHUNK)], soB)
        return carry

    lax.fori_loop(0, -(-NFULL // 8), chunk_body, 0)

    pltpu.make_async_copy(out_v.at[pl.ds(0, 128)],
                          z_hbm.at[q, pl.ds(0, 128), pl.ds(0, VCHUNK)],
                          soA).wait()
    pltpu.make_async_copy(out_v.at[pl.ds(128, ZCOLS - 128)],
                          z_hbm.at[q, pl.ds(128, ZCOLS - 128),
                                   pl.ds(0, VCHUNK)],
                          soB).wait()

    @pl.when(g == 7)
    def _():
        v0 = pl.multiple_of(NFULL * VCHUNK, VCHUNK)
        ob = out_v
        handles = [None, None]
        handles[0] = start(v0, 0, 0, VTAIL)
        for r in range(NRINGS):
            b = r & 1
            if r + 1 < NRINGS:
                handles[1 - b] = start(v0, r + 1, 1 - b, VTAIL)
            handles[b][0].wait()
            handles[b][1].wait()
            compute_r(r, b, ob, 1)
        center_cols(v0, ob, 1)
        pltpu.sync_copy(out_v, z_hbm.at[q, :, pl.ds(v0, VCHUNK)])


@jax.jit
def _sc_gather(y_q, ctr_t, w_t):
    mesh = plsc.VectorSubcoreMesh(core_axis_name="c", subcore_axis_name="s")
    f = pl.kernel(
        _sc_gather_body,
        out_type=jax.ShapeDtypeStruct((4, ZCOLS, NVPAD), jnp.float32),
        mesh=mesh,
        scratch_types=[
            pltpu.VMEM((4, NV), jnp.float32),
            pltpu.VMEM((2, 3, 8, VCHUNK), jnp.int32),
            pltpu.VMEM((2, 3, 8, VCHUNK), jnp.float32),
            pltpu.VMEM((ZCOLS, VCHUNK), jnp.float32),
            pltpu.SemaphoreType.DMA,
            pltpu.SemaphoreType.DMA,
            pltpu.SemaphoreType.DMA,
            pltpu.SemaphoreType.DMA,
            pltpu.SemaphoreType.DMA,
            pltpu.SemaphoreType.DMA,
        ],
        compiler_params=pltpu.CompilerParams(needs_layout_passes=False),
    )
    return f(y_q, ctr_t, w_t)


V_BLK = 1280


def _tc_body(z_ref, kb_ref, bb_ref, ob_ref):
    zb = z_ref[...]
    kb = kb_ref[...]
    acc = jnp.dot(kb[0], zb[0], preferred_element_type=jnp.float32)
    for qq in range(1, 4):
        acc = acc + jnp.dot(kb[qq], zb[qq],
                            preferred_element_type=jnp.float32)
    m = acc[0:16]
    for dd in range(1, 8):
        m = jnp.maximum(m, acc[dd * 16:(dd + 1) * 16])
    ob_ref[...] = jnp.maximum(m + bb_ref[...], 0.0)


@jax.jit
def _tc_conv(z, KbigT, bias2):
    grid = (NVPAD // V_BLK,)
    return pl.pallas_call(
        _tc_body,
        grid=grid,
        in_specs=[
            pl.BlockSpec((4, ZCOLS, V_BLK), lambda i: (0, 0, i)),
            pl.BlockSpec((4, 128, ZCOLS), lambda i: (0, 0, 0)),
            pl.BlockSpec((NFILTERS, 1), lambda i: (0, 0)),
        ],
        out_specs=pl.BlockSpec((NFILTERS, V_BLK), lambda i: (0, i)),
        out_shape=jax.ShapeDtypeStruct((NFILTERS, NVPAD), jnp.float32),
    )(z, KbigT, bias2)


def kernel(y, contributors, weights, angles, kernel, center_kernel, bias):
    del angles

    y_q = jnp.transpose(y, (0, 2, 1)).reshape(4, 4, NV)
    ctr_t = jnp.transpose(contributors, (0, 2, 4, 3, 1)).reshape(24, 8, NV)
    w_t = jnp.transpose(weights, (0, 2, 4, 3, 1)).reshape(24, 8, NV)

    Kb = jnp.stack([jnp.roll(kernel, dd, axis=1) for dd in range(NDIRS)],
                   axis=-2)
    Kb = Kb.reshape(NRINGS, NDIRS, 4, 4, NDIRS * NFILTERS)
    Kb = jnp.transpose(Kb, (2, 0, 1, 3, 4)).reshape(4, 256, 128)
    ckt = jnp.tile(center_kernel.reshape(4, 4, 1, NFILTERS),
                   (1, 1, NDIRS, 1)).reshape(4, 4, 128)
    Kbig = jnp.concatenate([Kb, ckt], axis=1)
    KbigT = jnp.transpose(Kbig, (0, 2, 1))

    z = _sc_gather(y_q, ctr_t, w_t)
    outT = _tc_conv(z, KbigT, bias.reshape(NFILTERS, 1))
    return jnp.transpose(outT[:, :NV])[None]


# speedup vs baseline: 1.1200x; 1.0077x over previous
import jax
import jax.numpy as jnp
from jax import lax
from jax.experimental import pallas as pl
from jax.experimental.pallas import tpu as pltpu
from jax.experimental.pallas import tpu_sc as plsc

NV = 10000
NRINGS = 8
NDIRS = 8
NCH = 16
NFILTERS = 16

VCHUNK = 128
NFULL = NV // VCHUNK
VTAIL = NV - NFULL * VCHUNK
ZCOLS = 260
NVPAD = 10240


def _sc_gather_body(yq_hbm, ctr_hbm, w_hbm, z_hbm, y_v, ctr_v, w_v, out_v,
                    sc0, sc1, sw0, sw1, soA, soB):
    h = lax.axis_index("c")
    s = lax.axis_index("s")
    q = 2 * h + lax.rem(s, 2)
    g = lax.div(s, 2)

    pltpu.sync_copy(yq_hbm.at[q], y_v)

    rowq = [jnp.broadcast_to(jnp.int32(ct), (16,)) for ct in range(4)]
    semc = [sc0, sc1]
    semw = [sw0, sw1]

    def start(v0, r, b, L=VCHUNK):
        if L == VCHUNK:
            cdst, wdst = ctr_v.at[b], w_v.at[b]
        else:
            cdst = ctr_v.at[b, :, :, pl.ds(0, L)]
            wdst = w_v.at[b, :, :, pl.ds(0, L)]
        hc = pltpu.async_copy(
            ctr_hbm.at[pl.ds(3 * r, 3), :, pl.ds(v0, L)], cdst, semc[b])
        hw = pltpu.async_copy(
            w_hbm.at[pl.ds(3 * r, 3), :, pl.ds(v0, L)], wdst, semw[b])
        return hc, hw

    def compute_r(r, b, ob, ngroups):
        @plsc.parallel_loop(0, 8 * ngroups, unroll=2)
        def _(dg):
            d = lax.div(dg, ngroups)
            gg = lax.rem(dg, ngroups)
            base = gg * 16
            c0 = ctr_v[b, 0, d, pl.ds(base, 16)]
            c1 = ctr_v[b, 1, d, pl.ds(base, 16)]
            c2 = ctr_v[b, 2, d, pl.ds(base, 16)]
            w0 = w_v[b, 0, d, pl.ds(base, 16)]
            w1 = w_v[b, 1, d, pl.ds(base, 16)]
            w2 = w_v[b, 2, d, pl.ds(base, 16)]
            colbase = r * 32 + d * 4
            for ct in range(4):
                g0 = plsc.load_gather(y_v, [rowq[ct], c0])
                g1 = plsc.load_gather(y_v, [rowq[ct], c1])
                g2 = plsc.load_gather(y_v, [rowq[ct], c2])
                acc = w0 * g0 + w1 * g1 + w2 * g2
                ob[colbase + ct, pl.ds(base, 16)] = acc

    def center_cols(v0, ob, ngroups):
        @plsc.parallel_loop(0, ngroups)
        def _(gg):
            for ct in range(4):
                ob[256 + ct, pl.ds(gg * 16, 16)] = (
                    y_v[ct, pl.ds(v0 + gg * 16, 16)])

    def chunk_body(t, carry):
        ci = g + 8 * t

        @pl.when(ci < NFULL)
        def _():
            v0 = pl.multiple_of(ci * VCHUNK, VCHUNK)

            @pl.when(t == 0)
            def _():
                hc, hw = start(v0, 0, 0)
                hc.wait()
                hw.wait()

            @pl.when(t > 0)
            def _():
                pltpu.make_async_copy(
                    ctr_hbm.at[pl.ds(0, 3), :, pl.ds(0, VCHUNK)],
                    ctr_v.at[0], semc[0]).wait()
                pltpu.make_async_copy(
                    w_hbm.at[pl.ds(0, 3), :, pl.ds(0, VCHUNK)],
                    w_v.at[0], semw[0]).wait()
                pltpu.make_async_copy(
                    out_v.at[pl.ds(0, 128)],
                    z_hbm.at[q, pl.ds(0, 128), pl.ds(0, VCHUNK)],
                    soA).wait()

            handles = [None, None]
            for r in range(NRINGS):
                b = r & 1
                if r + 1 < NRINGS:
                    handles[1 - b] = start(v0, r + 1, 1 - b)
                if r > 0:
                    handles[b][0].wait()
                    handles[b][1].wait()
                if r == 4:
                    @pl.when(t > 0)
                    def _():
                        pltpu.make_async_copy(
                            out_v.at[pl.ds(128, ZCOLS - 128)],
                            z_hbm.at[q, pl.ds(128, ZCOLS - 128),
                                     pl.ds(0, VCHUNK)],
                            soB).wait()
                if r == NRINGS - 1:
                    @pl.when(ci + 8 < NFULL)
                    def _():
                        start(pl.multiple_of((ci + 8) * VCHUNK, VCHUNK), 0, 0)
                compute_r(r, b, out_v, 8)
                if r == 3:
                    pltpu.async_copy(
                        out_v.at[pl.ds(0, 128)],
                        z_hbm.at[q, pl.ds(0, 128), pl.ds(v0, VCHUNK)], soA)

            center_cols(v0, out_v, 8)

            pltpu.async_copy(
                out_v.at[pl.ds(128, ZCOLS - 128)],
                z_hbm.at[q, pl.ds(128, ZCOLS - 128), pl.ds(v0, VCHUNK)], soB)
        return carry

    lax.fori_loop(0, -(-NFULL // 8), chunk_body, 0)

    pltpu.make_async_copy(out_v.at[pl.ds(0, 128)],
                          z_hbm.at[q, pl.ds(0, 128), pl.ds(0, VCHUNK)],
                          soA).wait()
    pltpu.make_async_copy(out_v.at[pl.ds(128, ZCOLS - 128)],
                          z_hbm.at[q, pl.ds(128, ZCOLS - 128),
                                   pl.ds(0, VCHUNK)],
                          soB).wait()

    @pl.when(g == 7)
    def _():
        v0 = pl.multiple_of(NFULL * VCHUNK, VCHUNK)
        ob = out_v
        handles = [None, None]
        handles[0] = start(v0, 0, 0, VTAIL)
        for r in range(NRINGS):
            b = r & 1
            if r + 1 < NRINGS:
                handles[1 - b] = start(v0, r + 1, 1 - b, VTAIL)
            handles[b][0].wait()
            handles[b][1].wait()
            compute_r(r, b, ob, 1)
        center_cols(v0, ob, 1)
        pltpu.sync_copy(out_v, z_hbm.at[q, :, pl.ds(v0, VCHUNK)])


@jax.jit
def _sc_gather(y_q, ctr_t, w_t):
    mesh = plsc.VectorSubcoreMesh(core_axis_name="c", subcore_axis_name="s")
    f = pl.kernel(
        _sc_gather_body,
        out_type=jax.ShapeDtypeStruct((4, ZCOLS, NVPAD), jnp.float32),
        mesh=mesh,
        scratch_types=[
            pltpu.VMEM((4, NV), jnp.float32),
            pltpu.VMEM((2, 3, 8, VCHUNK), jnp.int32),
            pltpu.VMEM((2, 3, 8, VCHUNK), jnp.float32),
            pltpu.VMEM((ZCOLS, VCHUNK), jnp.float32),
            pltpu.SemaphoreType.DMA,
            pltpu.SemaphoreType.DMA,
            pltpu.SemaphoreType.DMA,
            pltpu.SemaphoreType.DMA,
            pltpu.SemaphoreType.DMA,
            pltpu.SemaphoreType.DMA,
        ],
        compiler_params=pltpu.CompilerParams(needs_layout_passes=False),
    )
    return f(y_q, ctr_t, w_t)


V_BLK = 2560


def _tc_body(z_ref, kb_ref, bb_ref, ob_ref):
    zb = z_ref[...]
    kb = kb_ref[...]
    acc = jnp.dot(kb[0], zb[0], preferred_element_type=jnp.float32)
    for qq in range(1, 4):
        acc = acc + jnp.dot(kb[qq], zb[qq],
                            preferred_element_type=jnp.float32)
    m = acc[0:16]
    for dd in range(1, 8):
        m = jnp.maximum(m, acc[dd * 16:(dd + 1) * 16])
    ob_ref[...] = jnp.maximum(m + bb_ref[...], 0.0)


@jax.jit
def _tc_conv(z, KbigT, bias2):
    grid = (NVPAD // V_BLK,)
    return pl.pallas_call(
        _tc_body,
        grid=grid,
        in_specs=[
            pl.BlockSpec((4, ZCOLS, V_BLK), lambda i: (0, 0, i)),
            pl.BlockSpec((4, 128, ZCOLS), lambda i: (0, 0, 0)),
            pl.BlockSpec((NFILTERS, 1), lambda i: (0, 0)),
        ],
        out_specs=pl.BlockSpec((NFILTERS, V_BLK), lambda i: (0, i)),
        out_shape=jax.ShapeDtypeStruct((NFILTERS, NVPAD), jnp.float32),
    )(z, KbigT, bias2)


def kernel(y, contributors, weights, angles, kernel, center_kernel, bias):
    del angles

    y_q = jnp.transpose(y, (0, 2, 1)).reshape(4, 4, NV)
    ctr_t = jnp.transpose(contributors, (0, 2, 4, 3, 1)).reshape(24, 8, NV)
    w_t = jnp.transpose(weights, (0, 2, 4, 3, 1)).reshape(24, 8, NV)

    Kb = jnp.stack([jnp.roll(kernel, dd, axis=1) for dd in range(NDIRS)],
                   axis=-2)
    Kb = Kb.reshape(NRINGS, NDIRS, 4, 4, NDIRS * NFILTERS)
    Kb = jnp.transpose(Kb, (2, 0, 1, 3, 4)).reshape(4, 256, 128)
    ckt = jnp.tile(center_kernel.reshape(4, 4, 1, NFILTERS),
                   (1, 1, NDIRS, 1)).reshape(4, 4, 128)
    Kbig = jnp.concatenate([Kb, ckt], axis=1)
    KbigT = jnp.transpose(Kbig, (0, 2, 1))

    z = _sc_gather(y_q, ctr_t, w_t)
    outT = _tc_conv(z, KbigT, bias.reshape(NFILTERS, 1))
    return jnp.transpose(outT[:, :NV])[None]


# speedup vs baseline: 1.2325x; 1.1004x over previous
import jax
import jax.numpy as jnp
from jax import lax
from jax.experimental import pallas as pl
from jax.experimental.pallas import tpu as pltpu
from jax.experimental.pallas import tpu_sc as plsc

NV = 10000
NRINGS = 8
NDIRS = 8
NCH = 16
NFILTERS = 16

VCHUNK = 128
NFULL = NV // VCHUNK
VTAIL = NV - NFULL * VCHUNK
ZCOLS = 260
NVPAD = 10240


def _sc_gather_body(yq_hbm, ctr_hbm, w_hbm, z_hbm, y_v, ctr_v, w_v, out_v,
                    sc0, sc1, sw0, sw1, soA, soB):
    h = lax.axis_index("c")
    s = lax.axis_index("s")
    q = 2 * h + lax.rem(s, 2)
    g = lax.div(s, 2)

    pltpu.sync_copy(yq_hbm.at[q], y_v)

    rowq = [jnp.broadcast_to(jnp.int32(ct), (16,)) for ct in range(4)]
    semc = [sc0, sc1]
    semw = [sw0, sw1]

    def start(v0, p, b, L=VCHUNK):
        if L == VCHUNK:
            cdst, wdst = ctr_v.at[b], w_v.at[b]
        else:
            cdst = ctr_v.at[b, :, :, pl.ds(0, L)]
            wdst = w_v.at[b, :, :, pl.ds(0, L)]
        hc = pltpu.async_copy(
            ctr_hbm.at[pl.ds(6 * p, 6), :, pl.ds(v0, L)], cdst, semc[b])
        hw = pltpu.async_copy(
            w_hbm.at[pl.ds(6 * p, 6), :, pl.ds(v0, L)], wdst, semw[b])
        return hc, hw

    def compute_pair(p, b, ob, ngroups):
        @plsc.parallel_loop(0, 2 * 8 * ngroups, unroll=2)
        def _(dg):
            rr = lax.div(dg, 8 * ngroups)
            rem = lax.rem(dg, 8 * ngroups)
            d = lax.div(rem, ngroups)
            gg = lax.rem(rem, ngroups)
            base = gg * 16
            k0 = 3 * rr
            c0 = ctr_v[b, k0, d, pl.ds(base, 16)]
            c1 = ctr_v[b, k0 + 1, d, pl.ds(base, 16)]
            c2 = ctr_v[b, k0 + 2, d, pl.ds(base, 16)]
            w0 = w_v[b, k0, d, pl.ds(base, 16)]
            w1 = w_v[b, k0 + 1, d, pl.ds(base, 16)]
            w2 = w_v[b, k0 + 2, d, pl.ds(base, 16)]
            colbase = p * 64 + rr * 32 + d * 4
            for ct in range(4):
                g0 = plsc.load_gather(y_v, [rowq[ct], c0])
                g1 = plsc.load_gather(y_v, [rowq[ct], c1])
                g2 = plsc.load_gather(y_v, [rowq[ct], c2])
                acc = w0 * g0 + w1 * g1 + w2 * g2
                ob[colbase + ct, pl.ds(base, 16)] = acc

    def center_cols(v0, ob, ngroups):
        @plsc.parallel_loop(0, ngroups)
        def _(gg):
            for ct in range(4):
                ob[256 + ct, pl.ds(gg * 16, 16)] = (
                    y_v[ct, pl.ds(v0 + gg * 16, 16)])

    def chunk_body(t, carry):
        ci = g + 8 * t

        @pl.when(ci < NFULL)
        def _():
            v0 = pl.multiple_of(ci * VCHUNK, VCHUNK)

            @pl.when(t == 0)
            def _():
                hc, hw = start(v0, 0, 0)
                hc.wait()
                hw.wait()

            @pl.when(t > 0)
            def _():
                pltpu.make_async_copy(
                    ctr_hbm.at[pl.ds(0, 6), :, pl.ds(0, VCHUNK)],
                    ctr_v.at[0], semc[0]).wait()
                pltpu.make_async_copy(
                    w_hbm.at[pl.ds(0, 6), :, pl.ds(0, VCHUNK)],
                    w_v.at[0], semw[0]).wait()
                pltpu.make_async_copy(
                    out_v.at[pl.ds(0, 128)],
                    z_hbm.at[q, pl.ds(0, 128), pl.ds(0, VCHUNK)],
                    soA).wait()

            handles = [None, None]
            for p in range(NRINGS // 2):
                b = p & 1
                if p + 1 < NRINGS // 2:
                    handles[1 - b] = start(v0, p + 1, 1 - b)
                if p > 0:
                    handles[b][0].wait()
                    handles[b][1].wait()
                if p == 2:
                    @pl.when(t > 0)
                    def _():
                        pltpu.make_async_copy(
                            out_v.at[pl.ds(128, ZCOLS - 128)],
                            z_hbm.at[q, pl.ds(128, ZCOLS - 128),
                                     pl.ds(0, VCHUNK)],
                            soB).wait()
                if p == NRINGS // 2 - 1:
                    @pl.when(ci + 8 < NFULL)
                    def _():
                        start(pl.multiple_of((ci + 8) * VCHUNK, VCHUNK), 0, 0)
                compute_pair(p, b, out_v, 8)
                if p == 1:
                    pltpu.async_copy(
                        out_v.at[pl.ds(0, 128)],
                        z_hbm.at[q, pl.ds(0, 128), pl.ds(v0, VCHUNK)], soA)

            center_cols(v0, out_v, 8)

            pltpu.async_copy(
                out_v.at[pl.ds(128, ZCOLS - 128)],
                z_hbm.at[q, pl.ds(128, ZCOLS - 128), pl.ds(v0, VCHUNK)], soB)
        return carry

    lax.fori_loop(0, -(-NFULL // 8), chunk_body, 0)

    pltpu.make_async_copy(out_v.at[pl.ds(0, 128)],
                          z_hbm.at[q, pl.ds(0, 128), pl.ds(0, VCHUNK)],
                          soA).wait()
    pltpu.make_async_copy(out_v.at[pl.ds(128, ZCOLS - 128)],
                          z_hbm.at[q, pl.ds(128, ZCOLS - 128),
                                   pl.ds(0, VCHUNK)],
                          soB).wait()

    @pl.when(g == 7)
    def _():
        v0 = pl.multiple_of(NFULL * VCHUNK, VCHUNK)
        ob = out_v
        handles = [None, None]
        handles[0] = start(v0, 0, 0, VTAIL)
        for p in range(NRINGS // 2):
            b = p & 1
            if p + 1 < NRINGS // 2:
                handles[1 - b] = start(v0, p + 1, 1 - b, VTAIL)
            handles[b][0].wait()
            handles[b][1].wait()
            compute_pair(p, b, ob, 1)
        center_cols(v0, ob, 1)
        pltpu.sync_copy(out_v, z_hbm.at[q, :, pl.ds(v0, VCHUNK)])


@jax.jit
def _sc_gather(y_q, ctr_t, w_t):
    mesh = plsc.VectorSubcoreMesh(core_axis_name="c", subcore_axis_name="s")
    f = pl.kernel(
        _sc_gather_body,
        out_type=jax.ShapeDtypeStruct((4, ZCOLS, NVPAD), jnp.float32),
        mesh=mesh,
        scratch_types=[
            pltpu.VMEM((4, NV), jnp.float32),
            pltpu.VMEM((2, 6, 8, VCHUNK), jnp.int32),
            pltpu.VMEM((2, 6, 8, VCHUNK), jnp.float32),
            pltpu.VMEM((ZCOLS, VCHUNK), jnp.float32),
            pltpu.SemaphoreType.DMA,
            pltpu.SemaphoreType.DMA,
            pltpu.SemaphoreType.DMA,
            pltpu.SemaphoreType.DMA,
            pltpu.SemaphoreType.DMA,
            pltpu.SemaphoreType.DMA,
        ],
        compiler_params=pltpu.CompilerParams(needs_layout_passes=False),
    )
    return f(y_q, ctr_t, w_t)


V_BLK = 2560


def _tc_body(z_ref, kb_ref, bb_ref, ob_ref):
    zb = z_ref[...]
    kb = kb_ref[...]
    acc = jnp.dot(kb[0], zb[0], preferred_element_type=jnp.float32)
    for qq in range(1, 4):
        acc = acc + jnp.dot(kb[qq], zb[qq],
                            preferred_element_type=jnp.float32)
    m = acc[0:16]
    for dd in range(1, 8):
        m = jnp.maximum(m, acc[dd * 16:(dd + 1) * 16])
    ob_ref[...] = jnp.maximum(m + bb_ref[...], 0.0)


@jax.jit
def _tc_conv(z, KbigT, bias2):
    grid = (NVPAD // V_BLK,)
    return pl.pallas_call(
        _tc_body,
        grid=grid,
        in_specs=[
            pl.BlockSpec((4, ZCOLS, V_BLK), lambda i: (0, 0, i)),
            pl.BlockSpec((4, 128, ZCOLS), lambda i: (0, 0, 0)),
            pl.BlockSpec((NFILTERS, 1), lambda i: (0, 0)),
        ],
        out_specs=pl.BlockSpec((NFILTERS, V_BLK), lambda i: (0, i)),
        out_shape=jax.ShapeDtypeStruct((NFILTERS, NVPAD), jnp.float32),
    )(z, KbigT, bias2)


def kernel(y, contributors, weights, angles, kernel, center_kernel, bias):
    del angles

    y_q = jnp.transpose(y, (0, 2, 1)).reshape(4, 4, NV)
    ctr_t = jnp.transpose(contributors, (0, 2, 4, 3, 1)).reshape(24, 8, NV)
    w_t = jnp.transpose(weights, (0, 2, 4, 3, 1)).reshape(24, 8, NV)

    Kb = jnp.stack([jnp.roll(kernel, dd, axis=1) for dd in range(NDIRS)],
                   axis=-2)
    Kb = Kb.reshape(NRINGS, NDIRS, 4, 4, NDIRS * NFILTERS)
    Kb = jnp.transpose(Kb, (2, 0, 1, 3, 4)).reshape(4, 256, 128)
    ckt = jnp.tile(center_kernel.reshape(4, 4, 1, NFILTERS),
                   (1, 1, NDIRS, 1)).reshape(4, 4, 128)
    Kbig = jnp.concatenate([Kb, ckt], axis=1)
    KbigT = jnp.transpose(Kbig, (0, 2, 1))

    z = _sc_gather(y_q, ctr_t, w_t)
    outT = _tc_conv(z, KbigT, bias.reshape(NFILTERS, 1))
    return jnp.transpose(outT[:, :NV])[None]
